# Initial kernel scaffold; baseline (speedup 1.0000x reference)
import jax
import jax.numpy as jnp
from jax.experimental import pallas as pl


def kernel(batch, emb_weight):
    raise NotImplementedError("write your pallas kernel here")


# speedup vs baseline: 1.5613x; 1.5613x over previous
import jax
import jax.numpy as jnp
from jax import lax
from jax.experimental import pallas as pl
from jax.experimental.pallas import tpu as pltpu, tpu_sc as plsc

EMBED_DIM = 32


def _make_gather(V, D, B):
    info = plsc.get_sparse_core_info()
    NC, NS = info.num_cores, info.num_subcores
    NW = NC * NS
    assert B % NW == 0
    b_per_w = B // NW
    CH = 1664
    assert b_per_w % CH == 0 and CH % 8 == 0
    n_chunks = b_per_w // CH

    mesh = plsc.VectorSubcoreMesh(core_axis_name="c", subcore_axis_name="s")

    @jax.jit
    def run(table, idx):
        @pl.kernel(
            out_type=jax.ShapeDtypeStruct((B, D), jnp.float32),
            mesh=mesh,
            scratch_types=[
                pltpu.VMEM((CH,), jnp.int32),
                pltpu.VMEM((CH, D), jnp.float32),
                pltpu.SemaphoreType.DMA,
            ],
            compiler_params=pltpu.CompilerParams(use_tc_tiling_on_sc=False),
        )
        def k(table_hbm, idx_hbm, out_hbm, idx_v, rows_v, sem):
            wid = lax.axis_index("s") * NC + lax.axis_index("c")
            w_base = wid * b_per_w

            def body(i, _):
                base = w_base + i * CH
                pltpu.sync_copy(idx_hbm.at[pl.ds(base, CH)], idx_v)
                pltpu.async_copy(table_hbm.at[idx_v], rows_v, sem).wait()
                pltpu.sync_copy(rows_v, out_hbm.at[pl.ds(base, CH)])
                return 0

            lax.fori_loop(0, n_chunks, body, 0)

        return k(table, idx)

    return run


def kernel(batch, emb_weight):
    Bb, F = batch.shape
    B = Bb * F
    idx = batch.reshape(B).astype(jnp.int32)
    run = _make_gather(emb_weight.shape[0], emb_weight.shape[1], B)
    out = run(emb_weight, idx)
    return out.reshape(Bb, F, EMBED_DIM)


# speedup vs baseline: 1.5672x; 1.0038x over previous
import jax
import jax.numpy as jnp
from jax import lax
from jax.experimental import pallas as pl
from jax.experimental.pallas import tpu as pltpu, tpu_sc as plsc

EMBED_DIM = 32


def _make_gather(V, D, B):
    info = plsc.get_sparse_core_info()
    NC, NS = info.num_cores, info.num_subcores
    NW = NC * NS
    assert B % NW == 0
    b_per_w = B // NW
    CH = 1664
    NBUF = 2
    assert b_per_w % CH == 0 and CH % 8 == 0
    n_chunks = b_per_w // CH

    mesh = plsc.VectorSubcoreMesh(core_axis_name="c", subcore_axis_name="s")

    @jax.jit
    def run(table, idx):
        @pl.kernel(
            out_type=jax.ShapeDtypeStruct((B, D), jnp.float32),
            mesh=mesh,
            scratch_types=[
                pltpu.VMEM((b_per_w,), jnp.int32),
                pltpu.VMEM((NBUF, CH, D), jnp.float32),
                [pltpu.SemaphoreType.DMA] * NBUF,
                [pltpu.SemaphoreType.DMA] * NBUF,
            ],
            compiler_params=pltpu.CompilerParams(use_tc_tiling_on_sc=False),
        )
        def k(table_hbm, idx_hbm, out_hbm, idx_v, rows_v, gsems, wsems):
            wid = lax.axis_index("s") * NC + lax.axis_index("c")
            w_base = wid * b_per_w
            pltpu.sync_copy(idx_hbm.at[pl.ds(w_base, b_per_w)], idx_v)

            gathers = [None] * n_chunks
            writes = [None] * n_chunks
            for i in range(n_chunks):
                b = i % NBUF
                if i >= NBUF:
                    writes[i - NBUF].wait()
                gathers[i] = pltpu.async_copy(
                    table_hbm.at[idx_v.at[pl.ds(i * CH, CH)]],
                    rows_v.at[b],
                    gsems[b],
                )
                gathers[i].wait()
                writes[i] = pltpu.async_copy(
                    rows_v.at[b],
                    out_hbm.at[pl.ds(w_base + i * CH, CH)],
                    wsems[b],
                )
            for i in range(n_chunks - NBUF, n_chunks):
                writes[i].wait()

        return k(table, idx)

    return run


def kernel(batch, emb_weight):
    Bb, F = batch.shape
    B = Bb * F
    idx = batch.reshape(B).astype(jnp.int32)
    run = _make_gather(emb_weight.shape[0], emb_weight.shape[1], B)
    out = run(emb_weight, idx)
    return out.reshape(Bb, F, EMBED_DIM)
